# baseline (speedup 1.0000x reference)
import jax
import jax.numpy as jnp
from jax.experimental import pallas as pl
from jax.experimental.pallas import tpu as pltpu

_POS_MARGIN = 0.1
_NEG_MARGIN = 1.4
_LOG_SCALE = 24.0
_MAX_POINTS = 256
_M = 1024
_N = 1024
_K = 4096
_D = 512
_NPTS = 20000
_PPAD = 20480
_PCH = 2048
_NPCH = _PPAD // _PCH
_BPAD = 10240
_BCH = 1024
_CCH = 256
_GCH = 512
_HI = jax.lax.Precision.HIGHEST


def _nn_body(q_ref, p_ref, idx_ref, run_v, run_i):
    ch = pl.program_id(1)
    q = q_ref[0]
    p = p_ref[0]
    a2 = jnp.sum(q * q, axis=1)[:, None]
    b2 = jnp.sum(p * p, axis=0)[None, :]
    s = a2 + b2 - 2.0 * jnp.dot(q, p, preferred_element_type=jnp.float32)
    s = jnp.sqrt(jnp.clip(s, 0.0, None))
    gidx = ch * _PCH + jax.lax.broadcasted_iota(jnp.int32, (_M, _PCH), 1)

    @pl.when(ch == 0)
    def _():
        run_v[...] = s
        run_i[...] = gidx

    @pl.when(ch > 0)
    def _():
        rv = run_v[...]
        better = s < rv
        run_v[...] = jnp.where(better, s, rv)
        run_i[...] = jnp.where(better, gidx, run_i[...])

    @pl.when(ch == _NPCH - 1)
    def _():
        rv = run_v[...]
        ri = run_i[...]
        m = jnp.min(rv, axis=1)
        mi = jnp.min(jnp.where(rv == m[:, None], ri, jnp.int32(2 ** 30)),
                     axis=1)
        idx_ref[0] = mi[None, :]


def _nn_argmin(queries, points_t):
    return pl.pallas_call(
        _nn_body,
        grid=(2, _NPCH),
        in_specs=[
            pl.BlockSpec((1, _M, 3), lambda g, c: (g, 0, 0)),
            pl.BlockSpec((1, 3, _PCH), lambda g, c: (g, 0, c)),
        ],
        out_specs=pl.BlockSpec((1, 1, _M), lambda g, c: (g, 0, 0)),
        out_shape=jax.ShapeDtypeStruct((2, 1, _M), jnp.int32),
        scratch_shapes=[
            pltpu.VMEM((_M, _PCH), jnp.float32),
            pltpu.VMEM((_M, _PCH), jnp.int32),
        ],
    )(queries, points_t)


def _dense_body(r_ref, st_ref, row_ref, col_ref):
    f = jnp.dot(r_ref[...], st_ref[...], preferred_element_type=jnp.float32)
    feat = jnp.sqrt(jnp.clip(2.0 - 2.0 * f, 0.0, None))
    e = jnp.exp(_LOG_SCALE * jnp.maximum(_NEG_MARGIN - feat, 0.0) ** 2)
    row_ref[...] = jnp.sum(e, axis=1)[None, :]
    col_ref[...] = jnp.sum(e, axis=0)[None, :]


def _dense_sums(ref_feats, src_feats_t):
    return pl.pallas_call(
        _dense_body,
        out_shape=[
            jax.ShapeDtypeStruct((1, _M), jnp.float32),
            jax.ShapeDtypeStruct((1, _N), jnp.float32),
        ],
    )(ref_feats, src_feats_t)


def _combine_body(idx_ref_, back_ref, xgr, ygr, vr, xgc, ygc, vc,
                  rf_ref, sf_ref, row_ref, col_ref, out_ref,
                  act_s, sel_s, sp_s, sn_s, accr_s, accc_s):
    def mask_for(g):
        idx_row = idx_ref_[g, 0:1, :]

        def body(ch, acc):
            b = back_ref[g, pl.ds(ch * _BCH, _BCH), 0:1]
            hit = jnp.max(jnp.where(idx_row == b, 1.0, 0.0), axis=0,
                          keepdims=True)
            return jnp.maximum(acc, hit)

        return jax.lax.fori_loop(0, _BPAD // _BCH, body,
                                 jnp.zeros((1, _M), jnp.float32))

    mask_src_row = mask_for(0)
    mask_ref_row = mask_for(1)

    c_col = xgc[...] * _N + ygc[...]
    v_col = vc[...]
    ar_col = jax.lax.broadcasted_iota(jnp.int32, (_K, 1), 0)
    riota_col = jax.lax.broadcasted_iota(jnp.int32, (_M, 1), 0)

    def l1(i, carry):
        sl = pl.ds(i * _CCH, _CCH)
        xc_row = xgr[0:1, sl]
        yc_row = ygr[0:1, sl]
        cc_row = xc_row * _N + yc_row
        ii_row = i * _CCH + jax.lax.broadcasted_iota(jnp.int32, (1, _CCH), 1)
        dup = jnp.max(jnp.where((c_col == cc_row) & (ar_col > ii_row),
                                1.0, 0.0), axis=0, keepdims=True)
        ohx_t = jnp.where(riota_col == xc_row, 1.0, 0.0)
        ohy_t = jnp.where(riota_col == yc_row, 1.0, 0.0)
        mr = jnp.dot(mask_ref_row, ohx_t,
                     preferred_element_type=jnp.float32, precision=_HI)
        ms = jnp.dot(mask_src_row, ohy_t,
                     preferred_element_type=jnp.float32, precision=_HI)
        act_s[0:1, sl] = jnp.where((mr > 0.5) & (ms > 0.5) & (dup < 0.5),
                                   1.0, 0.0)
        return carry

    jax.lax.fori_loop(0, _K // _CCH, l1, 0)
    act_row = act_s[...]
    count = jnp.sum(act_row)

    def l2(i, carry):
        sl = pl.ds(i * _CCH, _CCH)
        vc_row = vr[0:1, sl]
        cc_row = xgr[0:1, sl] * _N + ygr[0:1, sl]
        beats = jnp.where((v_col > vc_row) |
                          ((v_col == vc_row) & (c_col < cc_row)), 1.0, 0.0)
        rank = jnp.dot(act_row, beats,
                       preferred_element_type=jnp.float32, precision=_HI)
        actc = act_s[0:1, sl]
        sel_s[0:1, sl] = jnp.where(
            (actc > 0.5) & ((count <= float(_MAX_POINTS)) |
                            (rank < float(_MAX_POINTS))), 1.0, 0.0)
        return carry

    jax.lax.fori_loop(0, _K // _CCH, l2, 0)

    ident = jnp.where(
        jax.lax.broadcasted_iota(jnp.int32, (_GCH, _GCH), 0) ==
        jax.lax.broadcasted_iota(jnp.int32, (_GCH, _GCH), 1), 1.0, 0.0)
    riota_row = jax.lax.broadcasted_iota(jnp.int32, (1, _M), 1)

    def l3(i, carry):
        sl = pl.ds(i * _GCH, _GCH)
        xc_col = xgc[sl, 0:1]
        yc_col = ygc[sl, 0:1]
        ohx = jnp.where(xc_col == riota_row, 1.0, 0.0)
        ohy = jnp.where(yc_col == riota_row, 1.0, 0.0)
        rg = jnp.dot(ohx, rf_ref[...],
                     preferred_element_type=jnp.float32, precision=_HI)
        sg = jnp.dot(ohy, sf_ref[...],
                     preferred_element_type=jnp.float32, precision=_HI)
        dotp = jnp.sum(rg * sg, axis=1, keepdims=True)
        fc = jnp.sqrt(jnp.clip(2.0 - 2.0 * dotp, 0.0, None))
        sp = _LOG_SCALE * jnp.maximum(fc - _POS_MARGIN, 0.0) ** 2
        sn = _LOG_SCALE * jnp.maximum(_NEG_MARGIN - fc, 0.0) ** 2
        sp_s[0:1, sl] = jnp.sum(ident * sp, axis=0, keepdims=True)
        sn_s[0:1, sl] = jnp.sum(ident * sn, axis=0, keepdims=True)
        return carry

    jax.lax.fori_loop(0, _K // _GCH, l3, 0)

    sel_row = sel_s[...]
    sp_row = sp_s[...]
    g_shift = jnp.max(jnp.where(sel_row > 0.5, sp_row, 0.0))

    accr_s[...] = jnp.zeros((8, _M), jnp.float32)
    accc_s[...] = jnp.zeros((8, _N), jnp.float32)

    def l4(i, carry):
        sl = pl.ds(i * _GCH, _GCH)
        selc = sel_s[0:1, sl]
        spc = sp_s[0:1, sl]
        snc = sn_s[0:1, sl]
        wp = jnp.exp(jnp.where(selc > 0.5, spc - g_shift, -1e30))
        wn = selc * jnp.exp(snc)
        w = jnp.concatenate(
            [selc, wp, wn, jnp.zeros((5, _GCH), jnp.float32)], axis=0)
        xc_col = xgc[sl, 0:1]
        yc_col = ygc[sl, 0:1]
        ohx = jnp.where(xc_col == riota_row, 1.0, 0.0)
        ohy = jnp.where(yc_col == riota_row, 1.0, 0.0)
        accr_s[...] = accr_s[...] + jnp.dot(
            w, ohx, preferred_element_type=jnp.float32, precision=_HI)
        accc_s[...] = accc_s[...] + jnp.dot(
            w, ohy, preferred_element_type=jnp.float32, precision=_HI)
        return carry

    jax.lax.fori_loop(0, _K // _GCH, l4, 0)

    npos_r, sp_r, sn_r = accr_s[0:1, :], accr_s[1:2, :], accr_s[2:3, :]
    npos_c, sp_c, sn_c = accc_s[0:1, :], accc_s[1:2, :], accc_s[2:3, :]

    row_sum = row_ref[...]
    col_sum = col_ref[...]
    eg = jnp.exp(-g_shift)
    lse_neg_r = jnp.log(row_sum - sn_r + npos_r)
    lse_pos_r = g_shift + jnp.log((float(_N) - npos_r) * eg + sp_r)
    lse_neg_c = jnp.log(col_sum - sn_c + npos_c)
    lse_pos_c = g_shift + jnp.log((float(_M) - npos_c) * eg + sp_c)

    def softplus(x):
        return jnp.maximum(x, 0.0) + jnp.log(1.0 + jnp.exp(-jnp.abs(x)))

    loss_row = softplus(lse_pos_r + lse_neg_r) / _LOG_SCALE
    loss_col = softplus(lse_pos_c + lse_neg_c) / _LOG_SCALE
    loss1 = (jnp.sum(loss_row) / float(_M) + jnp.sum(loss_col) / float(_N)) / 2.0
    loss2 = (jnp.sum(mask_ref_row) + jnp.sum(mask_src_row)) / float(_M + _N)
    loss = loss1 + loss2

    oi = jax.lax.broadcasted_iota(jnp.int32, (1, 128), 1)
    out_ref[...] = jnp.where(oi == 0, loss,
                             jnp.where(oi == 1, loss1,
                                       jnp.where(oi == 2, loss2, 0.0)))


def _combine(idx2, backs_t, xg_r, yg_r, v_r, xg_c, yg_c, v_c, rf, sf,
             row_sum, col_sum):
    return pl.pallas_call(
        _combine_body,
        out_shape=jax.ShapeDtypeStruct((1, 128), jnp.float32),
        scratch_shapes=[
            pltpu.VMEM((1, _K), jnp.float32),
            pltpu.VMEM((1, _K), jnp.float32),
            pltpu.VMEM((1, _K), jnp.float32),
            pltpu.VMEM((1, _K), jnp.float32),
            pltpu.VMEM((8, _M), jnp.float32),
            pltpu.VMEM((8, _N), jnp.float32),
        ],
    )(idx2, backs_t, xg_r, yg_r, v_r, xg_c, yg_c, v_c, rf, sf,
      row_sum, col_sum)


@jax.jit
def kernel(src_points, src_points_c, ref_points, ref_points_c, ref_feats_c,
           src_feats_c, gt_node_corr_indices, gt_node_corr_overlaps,
           src_back_indices, ref_back_indices):
    queries = jnp.stack([src_points_c, ref_points_c])
    pts = jnp.stack([src_points, ref_points])
    pts = jnp.pad(pts, ((0, 0), (0, _PPAD - _NPTS), (0, 0)),
                  constant_values=1e8)
    pts_t = jnp.transpose(pts, (0, 2, 1))
    idx2 = _nn_argmin(queries, pts_t)

    row_sum, col_sum = _dense_sums(ref_feats_c,
                                   jnp.transpose(src_feats_c, (1, 0)))

    backs = jnp.stack([src_back_indices, ref_back_indices])
    backs_t = jnp.pad(backs, ((0, 0), (0, _BPAD - backs.shape[1])),
                      constant_values=-1)[:, :, None]

    xg = gt_node_corr_indices[:, 0].astype(jnp.int32)
    yg = gt_node_corr_indices[:, 1].astype(jnp.int32)
    v = gt_node_corr_overlaps

    out = _combine(idx2, backs_t, xg[None, :], yg[None, :], v[None, :],
                   xg[:, None], yg[:, None], v[:, None],
                   ref_feats_c, src_feats_c, row_sum, col_sum)
    return (out[0, 0], out[0, 1], out[0, 2])


# speedup vs baseline: 1.1235x; 1.1235x over previous
import jax
import jax.numpy as jnp
from jax.experimental import pallas as pl
from jax.experimental.pallas import tpu as pltpu

_POS_MARGIN = 0.1
_NEG_MARGIN = 1.4
_LOG_SCALE = 24.0
_MAX_POINTS = 256
_M = 1024
_N = 1024
_K = 4096
_D = 512
_NPTS = 20000
_PPAD = 20480
_PCH = 2048
_NPCH = _PPAD // _PCH
_BPAD = 10240
_BCH = 1024
_CCH = 256
_GCH = 512
_HI = jax.lax.Precision.HIGHEST


def _nn_body(q_ref, p_ref, idx_ref, run_v, run_i):
    ch = pl.program_id(1)
    q = q_ref[0]
    p = p_ref[0]
    a2 = jnp.sum(q * q, axis=1)[:, None]
    b2 = jnp.sum(p * p, axis=0)[None, :]
    s = a2 + b2 - 2.0 * jnp.dot(q, p, preferred_element_type=jnp.float32)
    gidx = ch * _PCH + jax.lax.broadcasted_iota(jnp.int32, (_M, _PCH), 1)

    @pl.when(ch == 0)
    def _():
        run_v[...] = s
        run_i[...] = gidx

    @pl.when(ch > 0)
    def _():
        rv = run_v[...]
        better = s < rv
        run_v[...] = jnp.where(better, s, rv)
        run_i[...] = jnp.where(better, gidx, run_i[...])

    @pl.when(ch == _NPCH - 1)
    def _():
        rv = jnp.sqrt(jnp.clip(run_v[...], 0.0, None))
        ri = run_i[...]
        m = jnp.min(rv, axis=1)
        mi = jnp.min(jnp.where(rv == m[:, None], ri, jnp.int32(2 ** 30)),
                     axis=1)
        idx_ref[0] = mi[None, :]


def _nn_argmin(queries, points_t):
    return pl.pallas_call(
        _nn_body,
        grid=(2, _NPCH),
        in_specs=[
            pl.BlockSpec((1, _M, 3), lambda g, c: (g, 0, 0)),
            pl.BlockSpec((1, 3, _PCH), lambda g, c: (g, 0, c)),
        ],
        out_specs=pl.BlockSpec((1, 1, _M), lambda g, c: (g, 0, 0)),
        out_shape=jax.ShapeDtypeStruct((2, 1, _M), jnp.int32),
        scratch_shapes=[
            pltpu.VMEM((_M, _PCH), jnp.float32),
            pltpu.VMEM((_M, _PCH), jnp.int32),
        ],
    )(queries, points_t)


def _dense_body(r_ref, st_ref, row_ref, col_ref):
    f = jnp.dot(r_ref[...], st_ref[...], preferred_element_type=jnp.float32)
    feat = jnp.sqrt(jnp.clip(2.0 - 2.0 * f, 0.0, None))
    e = jnp.exp(_LOG_SCALE * jnp.maximum(_NEG_MARGIN - feat, 0.0) ** 2)
    row_ref[...] = jnp.sum(e, axis=1)[None, :]
    col_ref[...] = jnp.sum(e, axis=0)[None, :]


def _dense_sums(ref_feats, src_feats_t):
    return pl.pallas_call(
        _dense_body,
        out_shape=[
            jax.ShapeDtypeStruct((1, _M), jnp.float32),
            jax.ShapeDtypeStruct((1, _N), jnp.float32),
        ],
    )(ref_feats, src_feats_t)


def _combine_body(idx_ref_, back_ref, xgr, ygr, vr, xgc, ygc, vc,
                  rf_ref, sf_ref, row_ref, col_ref, out_ref,
                  act_s, sel_s, sp_s, sn_s, accr_s, accc_s):
    def mask_for(g):
        idx_row = idx_ref_[g, 0:1, :]

        def body(ch, acc):
            b = back_ref[g, pl.ds(ch * _BCH, _BCH), 0:1]
            hit = jnp.max(jnp.where(idx_row == b, 1.0, 0.0), axis=0,
                          keepdims=True)
            return jnp.maximum(acc, hit)

        return jax.lax.fori_loop(0, _BPAD // _BCH, body,
                                 jnp.zeros((1, _M), jnp.float32))

    mask_src_row = mask_for(0)
    mask_ref_row = mask_for(1)

    c_col = xgc[...] * _N + ygc[...]
    v_col = vc[...]
    ar_col = jax.lax.broadcasted_iota(jnp.int32, (_K, 1), 0)
    riota_col = jax.lax.broadcasted_iota(jnp.int32, (_M, 1), 0)

    def l1(i, carry):
        sl = pl.ds(i * _CCH, _CCH)
        xc_row = xgr[0:1, sl]
        yc_row = ygr[0:1, sl]
        cc_row = xc_row * _N + yc_row
        ii_row = i * _CCH + jax.lax.broadcasted_iota(jnp.int32, (1, _CCH), 1)
        dup = jnp.max(jnp.where((c_col == cc_row) & (ar_col > ii_row),
                                1.0, 0.0), axis=0, keepdims=True)
        ohx_t = jnp.where(riota_col == xc_row, 1.0, 0.0)
        ohy_t = jnp.where(riota_col == yc_row, 1.0, 0.0)
        mr = jnp.dot(mask_ref_row, ohx_t,
                     preferred_element_type=jnp.float32, precision=_HI)
        ms = jnp.dot(mask_src_row, ohy_t,
                     preferred_element_type=jnp.float32, precision=_HI)
        act_s[0:1, sl] = jnp.where((mr > 0.5) & (ms > 0.5) & (dup < 0.5),
                                   1.0, 0.0)
        return carry

    jax.lax.fori_loop(0, _K // _CCH, l1, 0)
    act_row = act_s[...]
    count = jnp.sum(act_row)

    def l2(i, carry):
        sl = pl.ds(i * _CCH, _CCH)
        vc_row = vr[0:1, sl]
        cc_row = xgr[0:1, sl] * _N + ygr[0:1, sl]
        beats = jnp.where((v_col > vc_row) |
                          ((v_col == vc_row) & (c_col < cc_row)), 1.0, 0.0)
        rank = jnp.dot(act_row, beats,
                       preferred_element_type=jnp.float32, precision=_HI)
        actc = act_s[0:1, sl]
        sel_s[0:1, sl] = jnp.where(
            (actc > 0.5) & ((count <= float(_MAX_POINTS)) |
                            (rank < float(_MAX_POINTS))), 1.0, 0.0)
        return carry

    jax.lax.fori_loop(0, _K // _CCH, l2, 0)

    ident = jnp.where(
        jax.lax.broadcasted_iota(jnp.int32, (_GCH, _GCH), 0) ==
        jax.lax.broadcasted_iota(jnp.int32, (_GCH, _GCH), 1), 1.0, 0.0)
    riota_row = jax.lax.broadcasted_iota(jnp.int32, (1, _M), 1)

    def l3(i, carry):
        sl = pl.ds(i * _GCH, _GCH)
        xc_col = xgc[sl, 0:1]
        yc_col = ygc[sl, 0:1]
        ohx = jnp.where(xc_col == riota_row, 1.0, 0.0)
        ohy = jnp.where(yc_col == riota_row, 1.0, 0.0)
        rg = jnp.dot(ohx, rf_ref[...],
                     preferred_element_type=jnp.float32, precision=_HI)
        sg = jnp.dot(ohy, sf_ref[...],
                     preferred_element_type=jnp.float32, precision=_HI)
        dotp = jnp.sum(rg * sg, axis=1, keepdims=True)
        fc = jnp.sqrt(jnp.clip(2.0 - 2.0 * dotp, 0.0, None))
        sp = _LOG_SCALE * jnp.maximum(fc - _POS_MARGIN, 0.0) ** 2
        sn = _LOG_SCALE * jnp.maximum(_NEG_MARGIN - fc, 0.0) ** 2
        sp_s[0:1, sl] = jnp.sum(ident * sp, axis=0, keepdims=True)
        sn_s[0:1, sl] = jnp.sum(ident * sn, axis=0, keepdims=True)
        return carry

    jax.lax.fori_loop(0, _K // _GCH, l3, 0)

    sel_row = sel_s[...]
    sp_row = sp_s[...]
    g_shift = jnp.max(jnp.where(sel_row > 0.5, sp_row, 0.0))

    accr_s[...] = jnp.zeros((8, _M), jnp.float32)
    accc_s[...] = jnp.zeros((8, _N), jnp.float32)

    def l4(i, carry):
        sl = pl.ds(i * _GCH, _GCH)
        selc = sel_s[0:1, sl]
        spc = sp_s[0:1, sl]
        snc = sn_s[0:1, sl]
        wp = jnp.exp(jnp.where(selc > 0.5, spc - g_shift, -1e30))
        wn = selc * jnp.exp(snc)
        w = jnp.concatenate(
            [selc, wp, wn, jnp.zeros((5, _GCH), jnp.float32)], axis=0)
        xc_col = xgc[sl, 0:1]
        yc_col = ygc[sl, 0:1]
        ohx = jnp.where(xc_col == riota_row, 1.0, 0.0)
        ohy = jnp.where(yc_col == riota_row, 1.0, 0.0)
        accr_s[...] = accr_s[...] + jnp.dot(
            w, ohx, preferred_element_type=jnp.float32, precision=_HI)
        accc_s[...] = accc_s[...] + jnp.dot(
            w, ohy, preferred_element_type=jnp.float32, precision=_HI)
        return carry

    jax.lax.fori_loop(0, _K // _GCH, l4, 0)

    npos_r, sp_r, sn_r = accr_s[0:1, :], accr_s[1:2, :], accr_s[2:3, :]
    npos_c, sp_c, sn_c = accc_s[0:1, :], accc_s[1:2, :], accc_s[2:3, :]

    row_sum = row_ref[...]
    col_sum = col_ref[...]
    eg = jnp.exp(-g_shift)
    lse_neg_r = jnp.log(row_sum - sn_r + npos_r)
    lse_pos_r = g_shift + jnp.log((float(_N) - npos_r) * eg + sp_r)
    lse_neg_c = jnp.log(col_sum - sn_c + npos_c)
    lse_pos_c = g_shift + jnp.log((float(_M) - npos_c) * eg + sp_c)

    def softplus(x):
        return jnp.maximum(x, 0.0) + jnp.log(1.0 + jnp.exp(-jnp.abs(x)))

    loss_row = softplus(lse_pos_r + lse_neg_r) / _LOG_SCALE
    loss_col = softplus(lse_pos_c + lse_neg_c) / _LOG_SCALE
    loss1 = (jnp.sum(loss_row) / float(_M) + jnp.sum(loss_col) / float(_N)) / 2.0
    loss2 = (jnp.sum(mask_ref_row) + jnp.sum(mask_src_row)) / float(_M + _N)
    loss = loss1 + loss2

    oi = jax.lax.broadcasted_iota(jnp.int32, (1, 128), 1)
    out_ref[...] = jnp.where(oi == 0, loss,
                             jnp.where(oi == 1, loss1,
                                       jnp.where(oi == 2, loss2, 0.0)))


def _combine(idx2, backs_t, xg_r, yg_r, v_r, xg_c, yg_c, v_c, rf, sf,
             row_sum, col_sum):
    return pl.pallas_call(
        _combine_body,
        out_shape=jax.ShapeDtypeStruct((1, 128), jnp.float32),
        scratch_shapes=[
            pltpu.VMEM((1, _K), jnp.float32),
            pltpu.VMEM((1, _K), jnp.float32),
            pltpu.VMEM((1, _K), jnp.float32),
            pltpu.VMEM((1, _K), jnp.float32),
            pltpu.VMEM((8, _M), jnp.float32),
            pltpu.VMEM((8, _N), jnp.float32),
        ],
    )(idx2, backs_t, xg_r, yg_r, v_r, xg_c, yg_c, v_c, rf, sf,
      row_sum, col_sum)


@jax.jit
def kernel(src_points, src_points_c, ref_points, ref_points_c, ref_feats_c,
           src_feats_c, gt_node_corr_indices, gt_node_corr_overlaps,
           src_back_indices, ref_back_indices):
    queries = jnp.stack([src_points_c, ref_points_c])
    pts = jnp.stack([src_points, ref_points])
    pts = jnp.pad(pts, ((0, 0), (0, _PPAD - _NPTS), (0, 0)),
                  constant_values=1e8)
    pts_t = jnp.transpose(pts, (0, 2, 1))
    idx2 = _nn_argmin(queries, pts_t)

    row_sum, col_sum = _dense_sums(ref_feats_c,
                                   jnp.transpose(src_feats_c, (1, 0)))

    backs = jnp.stack([src_back_indices, ref_back_indices])
    backs_t = jnp.pad(backs, ((0, 0), (0, _BPAD - backs.shape[1])),
                      constant_values=-1)[:, :, None]

    xg = gt_node_corr_indices[:, 0].astype(jnp.int32)
    yg = gt_node_corr_indices[:, 1].astype(jnp.int32)
    v = gt_node_corr_overlaps

    out = _combine(idx2, backs_t, xg[None, :], yg[None, :], v[None, :],
                   xg[:, None], yg[:, None], v[:, None],
                   ref_feats_c, src_feats_c, row_sum, col_sum)
    return (out[0, 0], out[0, 1], out[0, 2])


# speedup vs baseline: 1.5094x; 1.3435x over previous
import functools

import jax
import jax.numpy as jnp
from jax import lax
from jax.experimental import pallas as pl
from jax.experimental.pallas import tpu as pltpu
from jax.experimental.pallas import tpu_sc as plsc

_POS_MARGIN = 0.1
_NEG_MARGIN = 1.4
_LOG_SCALE = 24.0
_MAX_POINTS = 256
_M = 1024
_N = 1024
_K = 4096
_D = 512
_NPTS = 20000
_PPAD = 20480
_PCH = 2048
_NPCH = _PPAD // _PCH
_BPAD = 10240
_BCH = 1024
_CCH = 256
_GCH = 512
_HI = jax.lax.Precision.HIGHEST


def _nn_body(q_ref, p_ref, idx_ref, run_v, run_i):
    ch = pl.program_id(1)
    q = q_ref[0]
    p = p_ref[0]
    a2 = jnp.sum(q * q, axis=1)[:, None]
    b2 = jnp.sum(p * p, axis=0)[None, :]
    s = a2 + b2 - 2.0 * jnp.dot(q, p, preferred_element_type=jnp.float32)
    gidx = ch * _PCH + jax.lax.broadcasted_iota(jnp.int32, (_M, _PCH), 1)

    @pl.when(ch == 0)
    def _():
        run_v[...] = s
        run_i[...] = gidx

    @pl.when(ch > 0)
    def _():
        rv = run_v[...]
        better = s < rv
        run_v[...] = jnp.where(better, s, rv)
        run_i[...] = jnp.where(better, gidx, run_i[...])

    @pl.when(ch == _NPCH - 1)
    def _():
        rv = jnp.sqrt(jnp.clip(run_v[...], 0.0, None))
        ri = run_i[...]
        m = jnp.min(rv, axis=1)
        mi = jnp.min(jnp.where(rv == m[:, None], ri, jnp.int32(2 ** 30)),
                     axis=1)
        idx_ref[0] = mi[None, :]


def _nn_argmin(queries, points_t):
    return pl.pallas_call(
        _nn_body,
        grid=(2, _NPCH),
        in_specs=[
            pl.BlockSpec((1, _M, 3), lambda g, c: (g, 0, 0)),
            pl.BlockSpec((1, 3, _PCH), lambda g, c: (g, 0, c)),
        ],
        out_specs=pl.BlockSpec((1, 1, _M), lambda g, c: (g, 0, 0)),
        out_shape=jax.ShapeDtypeStruct((2, 1, _M), jnp.int32),
        scratch_shapes=[
            pltpu.VMEM((_M, _PCH), jnp.float32),
            pltpu.VMEM((_M, _PCH), jnp.int32),
        ],
    )(queries, points_t)


def _dense_body(r_ref, st_ref, row_ref, col_ref):
    f = jnp.dot(r_ref[...], st_ref[...], preferred_element_type=jnp.float32)
    feat = jnp.sqrt(jnp.clip(2.0 - 2.0 * f, 0.0, None))
    e = jnp.exp(_LOG_SCALE * jnp.maximum(_NEG_MARGIN - feat, 0.0) ** 2)
    row_ref[...] = jnp.sum(e, axis=1)[None, :]
    col_ref[...] = jnp.sum(e, axis=0)[None, :]


def _dense_sums(ref_feats, src_feats_t):
    return pl.pallas_call(
        _dense_body,
        out_shape=[
            jax.ShapeDtypeStruct((1, _M), jnp.float32),
            jax.ShapeDtypeStruct((1, _N), jnp.float32),
        ],
    )(ref_feats, src_feats_t)


_NW = 32
_BPW = _K // _NW
_HB = 64


def _cand_dots_sc(rf, sf, xg, yg):
    mesh = plsc.VectorSubcoreMesh(core_axis_name="c", subcore_axis_name="s")

    @functools.partial(
        pl.kernel, mesh=mesh,
        out_type=jax.ShapeDtypeStruct((_K,), jnp.float32),
        scratch_types=[
            pltpu.VMEM((_BPW,), jnp.int32),
            pltpu.VMEM((_BPW,), jnp.int32),
            pltpu.VMEM((_HB, _D), jnp.float32),
            pltpu.VMEM((_HB, _D), jnp.float32),
            pltpu.VMEM((_BPW,), jnp.float32),
            pltpu.SMEM((_BPW,), jnp.float32),
            pltpu.SemaphoreType.DMA,
        ],
    )
    def k(rf_hbm, sf_hbm, xg_hbm, yg_hbm, out_hbm, xi, yi, rrows, srows,
          dots, dots_sm, sem):
        wid = lax.axis_index("s") * 2 + lax.axis_index("c")
        base = wid * _BPW
        lane = lax.iota(jnp.int32, 16)
        pltpu.sync_copy(xg_hbm.at[pl.ds(base, _BPW)], xi)
        pltpu.sync_copy(yg_hbm.at[pl.ds(base, _BPW)], yi)
        for h in range(_BPW // _HB):
            pltpu.async_copy(rf_hbm.at[xi.at[pl.ds(h * _HB, _HB)]], rrows,
                             sem).wait()
            pltpu.async_copy(sf_hbm.at[yi.at[pl.ds(h * _HB, _HB)]], srows,
                             sem).wait()

            def row(r, carry):
                acc = jnp.zeros((16,), jnp.float32)
                for kk in range(_D // 16):
                    acc = acc + (rrows[r, pl.ds(kk * 16, 16)] *
                                 srows[r, pl.ds(kk * 16, 16)])
                s = acc[0]
                for l in range(1, 16):
                    s = s + acc[l]
                dots_sm[h * _HB + r] = s
                return carry

            lax.fori_loop(0, _HB, row, 0)
        for g in range(_BPW // 16):
            vec = jnp.zeros((16,), jnp.float32)
            for l in range(16):
                vec = jnp.where(lane == l, dots_sm[g * 16 + l], vec)
            dots[g * 16:(g + 1) * 16] = vec
        pltpu.sync_copy(dots, out_hbm.at[pl.ds(base, _BPW)])

    return k(rf, sf, xg, yg)


def _combine_body(idx_ref_, back_ref, xgr, ygr, vr, xgc, ygc, vc,
                  dots_ref, row_ref, col_ref, out_ref,
                  act_s, sel_s, sp_s, sn_s, accr_s, accc_s):
    def mask_for(g):
        idx_row = idx_ref_[g, 0:1, :]

        def body(ch, acc):
            b = back_ref[g, pl.ds(ch * _BCH, _BCH), 0:1]
            hit = jnp.max(jnp.where(idx_row == b, 1.0, 0.0), axis=0,
                          keepdims=True)
            return jnp.maximum(acc, hit)

        return jax.lax.fori_loop(0, _BPAD // _BCH, body,
                                 jnp.zeros((1, _M), jnp.float32))

    mask_src_row = mask_for(0)
    mask_ref_row = mask_for(1)

    c_col = xgc[...] * _N + ygc[...]
    v_col = vc[...]
    ar_col = jax.lax.broadcasted_iota(jnp.int32, (_K, 1), 0)
    riota_col = jax.lax.broadcasted_iota(jnp.int32, (_M, 1), 0)

    def l1(i, carry):
        sl = pl.ds(i * _CCH, _CCH)
        xc_row = xgr[0:1, sl]
        yc_row = ygr[0:1, sl]
        cc_row = xc_row * _N + yc_row
        ii_row = i * _CCH + jax.lax.broadcasted_iota(jnp.int32, (1, _CCH), 1)
        dup = jnp.max(jnp.where((c_col == cc_row) & (ar_col > ii_row),
                                1.0, 0.0), axis=0, keepdims=True)
        ohx_t = jnp.where(riota_col == xc_row, 1.0, 0.0)
        ohy_t = jnp.where(riota_col == yc_row, 1.0, 0.0)
        mr = jnp.dot(mask_ref_row, ohx_t, preferred_element_type=jnp.float32)
        ms = jnp.dot(mask_src_row, ohy_t, preferred_element_type=jnp.float32)
        act_s[0:1, sl] = jnp.where((mr > 0.5) & (ms > 0.5) & (dup < 0.5),
                                   1.0, 0.0)
        return carry

    jax.lax.fori_loop(0, _K // _CCH, l1, 0)
    act_row = act_s[...]
    count = jnp.sum(act_row)

    def l2(i, carry):
        sl = pl.ds(i * _CCH, _CCH)
        vc_row = vr[0:1, sl]
        cc_row = xgr[0:1, sl] * _N + ygr[0:1, sl]
        beats = jnp.where((v_col > vc_row) |
                          ((v_col == vc_row) & (c_col < cc_row)), 1.0, 0.0)
        rank = jnp.dot(act_row, beats, preferred_element_type=jnp.float32)
        actc = act_s[0:1, sl]
        sel_s[0:1, sl] = jnp.where(
            (actc > 0.5) & ((count <= float(_MAX_POINTS)) |
                            (rank < float(_MAX_POINTS))), 1.0, 0.0)
        return carry

    jax.lax.fori_loop(0, _K // _CCH, l2, 0)

    riota_row = jax.lax.broadcasted_iota(jnp.int32, (1, _M), 1)
    fc = jnp.sqrt(jnp.clip(2.0 - 2.0 * dots_ref[...], 0.0, None))
    sp_s[...] = _LOG_SCALE * jnp.maximum(fc - _POS_MARGIN, 0.0) ** 2
    sn_s[...] = _LOG_SCALE * jnp.maximum(_NEG_MARGIN - fc, 0.0) ** 2

    sel_row = sel_s[...]
    sp_row = sp_s[...]
    g_shift = jnp.max(jnp.where(sel_row > 0.5, sp_row, 0.0))

    accr_s[...] = jnp.zeros((8, _M), jnp.float32)
    accc_s[...] = jnp.zeros((8, _N), jnp.float32)

    def l4(i, carry):
        sl = pl.ds(i * _GCH, _GCH)
        selc = sel_s[0:1, sl]
        spc = sp_s[0:1, sl]
        snc = sn_s[0:1, sl]
        wp = jnp.exp(jnp.where(selc > 0.5, spc - g_shift, -1e30))
        wn = selc * jnp.exp(snc)
        w = jnp.concatenate(
            [selc, wp, wn, jnp.zeros((5, _GCH), jnp.float32)], axis=0)
        xc_col = xgc[sl, 0:1]
        yc_col = ygc[sl, 0:1]
        ohx = jnp.where(xc_col == riota_row, 1.0, 0.0)
        ohy = jnp.where(yc_col == riota_row, 1.0, 0.0)
        accr_s[...] = accr_s[...] + jnp.dot(
            w, ohx, preferred_element_type=jnp.float32, precision=_HI)
        accc_s[...] = accc_s[...] + jnp.dot(
            w, ohy, preferred_element_type=jnp.float32, precision=_HI)
        return carry

    jax.lax.fori_loop(0, _K // _GCH, l4, 0)

    npos_r, sp_r, sn_r = accr_s[0:1, :], accr_s[1:2, :], accr_s[2:3, :]
    npos_c, sp_c, sn_c = accc_s[0:1, :], accc_s[1:2, :], accc_s[2:3, :]

    row_sum = row_ref[...]
    col_sum = col_ref[...]
    eg = jnp.exp(-g_shift)
    lse_neg_r = jnp.log(row_sum - sn_r + npos_r)
    lse_pos_r = g_shift + jnp.log((float(_N) - npos_r) * eg + sp_r)
    lse_neg_c = jnp.log(col_sum - sn_c + npos_c)
    lse_pos_c = g_shift + jnp.log((float(_M) - npos_c) * eg + sp_c)

    def softplus(x):
        return jnp.maximum(x, 0.0) + jnp.log(1.0 + jnp.exp(-jnp.abs(x)))

    loss_row = softplus(lse_pos_r + lse_neg_r) / _LOG_SCALE
    loss_col = softplus(lse_pos_c + lse_neg_c) / _LOG_SCALE
    loss1 = (jnp.sum(loss_row) / float(_M) + jnp.sum(loss_col) / float(_N)) / 2.0
    loss2 = (jnp.sum(mask_ref_row) + jnp.sum(mask_src_row)) / float(_M + _N)
    loss = loss1 + loss2

    oi = jax.lax.broadcasted_iota(jnp.int32, (1, 128), 1)
    out_ref[...] = jnp.where(oi == 0, loss,
                             jnp.where(oi == 1, loss1,
                                       jnp.where(oi == 2, loss2, 0.0)))


def _combine(idx2, backs_t, xg_r, yg_r, v_r, xg_c, yg_c, v_c, dots,
             row_sum, col_sum):
    return pl.pallas_call(
        _combine_body,
        out_shape=jax.ShapeDtypeStruct((1, 128), jnp.float32),
        scratch_shapes=[
            pltpu.VMEM((1, _K), jnp.float32),
            pltpu.VMEM((1, _K), jnp.float32),
            pltpu.VMEM((1, _K), jnp.float32),
            pltpu.VMEM((1, _K), jnp.float32),
            pltpu.VMEM((8, _M), jnp.float32),
            pltpu.VMEM((8, _N), jnp.float32),
        ],
    )(idx2, backs_t, xg_r, yg_r, v_r, xg_c, yg_c, v_c, dots,
      row_sum, col_sum)


@jax.jit
def kernel(src_points, src_points_c, ref_points, ref_points_c, ref_feats_c,
           src_feats_c, gt_node_corr_indices, gt_node_corr_overlaps,
           src_back_indices, ref_back_indices):
    queries = jnp.stack([src_points_c, ref_points_c])
    pts = jnp.stack([src_points, ref_points])
    pts = jnp.pad(pts, ((0, 0), (0, _PPAD - _NPTS), (0, 0)),
                  constant_values=1e8)
    pts_t = jnp.transpose(pts, (0, 2, 1))
    idx2 = _nn_argmin(queries, pts_t)

    row_sum, col_sum = _dense_sums(ref_feats_c,
                                   jnp.transpose(src_feats_c, (1, 0)))

    backs = jnp.stack([src_back_indices, ref_back_indices])
    backs_t = jnp.pad(backs, ((0, 0), (0, _BPAD - backs.shape[1])),
                      constant_values=-1)[:, :, None]

    xg = gt_node_corr_indices[:, 0].astype(jnp.int32)
    yg = gt_node_corr_indices[:, 1].astype(jnp.int32)
    v = gt_node_corr_overlaps

    dots = _cand_dots_sc(ref_feats_c, src_feats_c, xg, yg)

    out = _combine(idx2, backs_t, xg[None, :], yg[None, :], v[None, :],
                   xg[:, None], yg[:, None], v[:, None],
                   dots[None, :], row_sum, col_sum)
    return (out[0, 0], out[0, 1], out[0, 2])


# speedup vs baseline: 1.5814x; 1.0477x over previous
import functools

import jax
import jax.numpy as jnp
from jax import lax
from jax.experimental import pallas as pl
from jax.experimental.pallas import tpu as pltpu
from jax.experimental.pallas import tpu_sc as plsc

_POS_MARGIN = 0.1
_NEG_MARGIN = 1.4
_LOG_SCALE = 24.0
_MAX_POINTS = 256
_M = 1024
_N = 1024
_K = 4096
_D = 512
_NPTS = 20000
_PPAD = 20480
_PCH = 2048
_NPCH = _PPAD // _PCH
_BPAD = 10240
_BCH = 1024
_CCH = 512
_GCH = 512
_HI = jax.lax.Precision.HIGHEST


def _nn_body(q_ref, p_ref, idx_ref, run_v, run_i):
    ch = pl.program_id(1)
    q = q_ref[0]
    p = p_ref[0]
    a2 = jnp.sum(q * q, axis=1)[:, None]
    b2 = jnp.sum(p * p, axis=0)[None, :]
    s = a2 + b2 - 2.0 * jnp.dot(q, p, preferred_element_type=jnp.float32)
    gidx = ch * _PCH + jax.lax.broadcasted_iota(jnp.int32, (_M, _PCH), 1)

    @pl.when(ch == 0)
    def _():
        run_v[...] = s
        run_i[...] = gidx

    @pl.when(ch > 0)
    def _():
        rv = run_v[...]
        better = s < rv
        run_v[...] = jnp.where(better, s, rv)
        run_i[...] = jnp.where(better, gidx, run_i[...])

    @pl.when(ch == _NPCH - 1)
    def _():
        rv = jnp.sqrt(jnp.clip(run_v[...], 0.0, None))
        ri = run_i[...]
        m = jnp.min(rv, axis=1)
        mi = jnp.min(jnp.where(rv == m[:, None], ri, jnp.int32(2 ** 30)),
                     axis=1)
        idx_ref[0] = mi[None, :]


def _nn_argmin(queries, points_t):
    return pl.pallas_call(
        _nn_body,
        grid=(2, _NPCH),
        in_specs=[
            pl.BlockSpec((1, _M, 3), lambda g, c: (g, 0, 0)),
            pl.BlockSpec((1, 3, _PCH), lambda g, c: (g, 0, c)),
        ],
        out_specs=pl.BlockSpec((1, 1, _M), lambda g, c: (g, 0, 0)),
        out_shape=jax.ShapeDtypeStruct((2, 1, _M), jnp.int32),
        scratch_shapes=[
            pltpu.VMEM((_M, _PCH), jnp.float32),
            pltpu.VMEM((_M, _PCH), jnp.int32),
        ],
    )(queries, points_t)


def _dense_body(r_ref, st_ref, row_ref, col_ref):
    f = jnp.dot(r_ref[...], st_ref[...], preferred_element_type=jnp.float32)
    feat = jnp.sqrt(jnp.clip(2.0 - 2.0 * f, 0.0, None))
    e = jnp.exp(_LOG_SCALE * jnp.maximum(_NEG_MARGIN - feat, 0.0) ** 2)
    row_ref[...] = jnp.sum(e, axis=1)[None, :]
    col_ref[...] = jnp.sum(e, axis=0)[None, :]


def _dense_sums(ref_feats, src_feats_t):
    return pl.pallas_call(
        _dense_body,
        out_shape=[
            jax.ShapeDtypeStruct((1, _M), jnp.float32),
            jax.ShapeDtypeStruct((1, _N), jnp.float32),
        ],
    )(ref_feats, src_feats_t)


_NW = 32
_BPW = _K // _NW
_HB = 64


def _cand_dots_sc(rf, sf, xg, yg):
    mesh = plsc.VectorSubcoreMesh(core_axis_name="c", subcore_axis_name="s")

    @functools.partial(
        pl.kernel, mesh=mesh,
        out_type=jax.ShapeDtypeStruct((_K,), jnp.float32),
        scratch_types=[
            pltpu.VMEM((_BPW,), jnp.int32),
            pltpu.VMEM((_BPW,), jnp.int32),
            pltpu.VMEM((_HB, _D), jnp.float32),
            pltpu.VMEM((_HB, _D), jnp.float32),
            pltpu.VMEM((_BPW,), jnp.float32),
            pltpu.SMEM((_BPW,), jnp.float32),
            pltpu.SemaphoreType.DMA,
        ],
    )
    def k(rf_hbm, sf_hbm, xg_hbm, yg_hbm, out_hbm, xi, yi, rrows, srows,
          dots, dots_sm, sem):
        wid = lax.axis_index("s") * 2 + lax.axis_index("c")
        base = wid * _BPW
        lane = lax.iota(jnp.int32, 16)
        pltpu.sync_copy(xg_hbm.at[pl.ds(base, _BPW)], xi)
        pltpu.sync_copy(yg_hbm.at[pl.ds(base, _BPW)], yi)
        for h in range(_BPW // _HB):
            pltpu.async_copy(rf_hbm.at[xi.at[pl.ds(h * _HB, _HB)]], rrows,
                             sem).wait()
            pltpu.async_copy(sf_hbm.at[yi.at[pl.ds(h * _HB, _HB)]], srows,
                             sem).wait()

            def row(r, carry):
                acc = jnp.zeros((16,), jnp.float32)
                for kk in range(_D // 16):
                    acc = acc + (rrows[r, pl.ds(kk * 16, 16)] *
                                 srows[r, pl.ds(kk * 16, 16)])
                s = acc[0]
                for l in range(1, 16):
                    s = s + acc[l]
                dots_sm[h * _HB + r] = s
                return carry

            lax.fori_loop(0, _HB, row, 0)
        for g in range(_BPW // 16):
            vec = jnp.zeros((16,), jnp.float32)
            for l in range(16):
                vec = jnp.where(lane == l, dots_sm[g * 16 + l], vec)
            dots[g * 16:(g + 1) * 16] = vec
        pltpu.sync_copy(dots, out_hbm.at[pl.ds(base, _BPW)])

    return k(rf, sf, xg, yg)


def _combine_body(idx_ref_, back_ref, xgr, ygr, vr, xgc, ygc, vc,
                  dots_ref, row_ref, col_ref, out_ref,
                  act_s, sel_s, sp_s, sn_s, accr_s, accc_s):
    def mask_for(g):
        idx_row = idx_ref_[g, 0:1, :]

        def body(ch, acc):
            b = back_ref[g, pl.ds(ch * _BCH, _BCH), 0:1]
            hit = jnp.max(jnp.where(idx_row == b, 1.0, 0.0), axis=0,
                          keepdims=True)
            return jnp.maximum(acc, hit)

        return jax.lax.fori_loop(0, _BPAD // _BCH, body,
                                 jnp.zeros((1, _M), jnp.float32))

    mask_src_row = mask_for(0)
    mask_ref_row = mask_for(1)

    c_col = xgc[...] * _N + ygc[...]
    v_col = vc[...]
    ar_col = jax.lax.broadcasted_iota(jnp.int32, (_K, 1), 0)
    riota_col = jax.lax.broadcasted_iota(jnp.int32, (_M, 1), 0)

    def l1(i, carry):
        sl = pl.ds(i * _CCH, _CCH)
        xc_row = xgr[0:1, sl]
        yc_row = ygr[0:1, sl]
        cc_row = xc_row * _N + yc_row
        ii_row = i * _CCH + jax.lax.broadcasted_iota(jnp.int32, (1, _CCH), 1)
        dup = jnp.max(jnp.where((c_col == cc_row) & (ar_col > ii_row),
                                1.0, 0.0), axis=0, keepdims=True)
        ohx_t = jnp.where(riota_col == xc_row, 1.0, 0.0)
        ohy_t = jnp.where(riota_col == yc_row, 1.0, 0.0)
        mr = jnp.dot(mask_ref_row, ohx_t, preferred_element_type=jnp.float32)
        ms = jnp.dot(mask_src_row, ohy_t, preferred_element_type=jnp.float32)
        act_s[0:1, sl] = jnp.where((mr > 0.5) & (ms > 0.5) & (dup < 0.5),
                                   1.0, 0.0)
        return carry

    jax.lax.fori_loop(0, _K // _CCH, l1, 0)
    act_row = act_s[...]
    count = jnp.sum(act_row)

    def l2(i, carry):
        sl = pl.ds(i * _CCH, _CCH)
        vc_row = vr[0:1, sl]
        cc_row = xgr[0:1, sl] * _N + ygr[0:1, sl]
        beats = jnp.where((v_col > vc_row) |
                          ((v_col == vc_row) & (c_col < cc_row)), 1.0, 0.0)
        rank = jnp.dot(act_row, beats, preferred_element_type=jnp.float32)
        actc = act_s[0:1, sl]
        sel_s[0:1, sl] = jnp.where(
            (actc > 0.5) & ((count <= float(_MAX_POINTS)) |
                            (rank < float(_MAX_POINTS))), 1.0, 0.0)
        return carry

    jax.lax.fori_loop(0, _K // _CCH, l2, 0)

    riota_row = jax.lax.broadcasted_iota(jnp.int32, (1, _M), 1)
    fc = jnp.sqrt(jnp.clip(2.0 - 2.0 * dots_ref[...], 0.0, None))
    sp_s[...] = _LOG_SCALE * jnp.maximum(fc - _POS_MARGIN, 0.0) ** 2
    sn_s[...] = _LOG_SCALE * jnp.maximum(_NEG_MARGIN - fc, 0.0) ** 2

    sel_row = sel_s[...]
    sp_row = sp_s[...]
    g_shift = jnp.max(jnp.where(sel_row > 0.5, sp_row, 0.0))

    accr_s[...] = jnp.zeros((8, _M), jnp.float32)
    accc_s[...] = jnp.zeros((8, _N), jnp.float32)

    def l4(i, carry):
        sl = pl.ds(i * _GCH, _GCH)
        selc = sel_s[0:1, sl]
        spc = sp_s[0:1, sl]
        snc = sn_s[0:1, sl]
        wp = jnp.exp(jnp.where(selc > 0.5, spc - g_shift, -1e30))
        wn = selc * jnp.exp(snc)
        w = jnp.concatenate(
            [selc, wp, wn, jnp.zeros((5, _GCH), jnp.float32)], axis=0)
        xc_col = xgc[sl, 0:1]
        yc_col = ygc[sl, 0:1]
        ohx = jnp.where(xc_col == riota_row, 1.0, 0.0)
        ohy = jnp.where(yc_col == riota_row, 1.0, 0.0)
        accr_s[...] = accr_s[...] + jnp.dot(
            w, ohx, preferred_element_type=jnp.float32, precision=_HI)
        accc_s[...] = accc_s[...] + jnp.dot(
            w, ohy, preferred_element_type=jnp.float32, precision=_HI)
        return carry

    jax.lax.fori_loop(0, _K // _GCH, l4, 0)

    npos_r, sp_r, sn_r = accr_s[0:1, :], accr_s[1:2, :], accr_s[2:3, :]
    npos_c, sp_c, sn_c = accc_s[0:1, :], accc_s[1:2, :], accc_s[2:3, :]

    row_sum = row_ref[...]
    col_sum = col_ref[...]
    eg = jnp.exp(-g_shift)
    lse_neg_r = jnp.log(row_sum - sn_r + npos_r)
    lse_pos_r = g_shift + jnp.log((float(_N) - npos_r) * eg + sp_r)
    lse_neg_c = jnp.log(col_sum - sn_c + npos_c)
    lse_pos_c = g_shift + jnp.log((float(_M) - npos_c) * eg + sp_c)

    def softplus(x):
        return jnp.maximum(x, 0.0) + jnp.log(1.0 + jnp.exp(-jnp.abs(x)))

    loss_row = softplus(lse_pos_r + lse_neg_r) / _LOG_SCALE
    loss_col = softplus(lse_pos_c + lse_neg_c) / _LOG_SCALE
    loss1 = (jnp.sum(loss_row) / float(_M) + jnp.sum(loss_col) / float(_N)) / 2.0
    loss2 = (jnp.sum(mask_ref_row) + jnp.sum(mask_src_row)) / float(_M + _N)
    loss = loss1 + loss2

    oi = jax.lax.broadcasted_iota(jnp.int32, (1, 128), 1)
    out_ref[...] = jnp.where(oi == 0, loss,
                             jnp.where(oi == 1, loss1,
                                       jnp.where(oi == 2, loss2, 0.0)))


def _combine(idx2, backs_t, xg_r, yg_r, v_r, xg_c, yg_c, v_c, dots,
             row_sum, col_sum):
    return pl.pallas_call(
        _combine_body,
        out_shape=jax.ShapeDtypeStruct((1, 128), jnp.float32),
        scratch_shapes=[
            pltpu.VMEM((1, _K), jnp.float32),
            pltpu.VMEM((1, _K), jnp.float32),
            pltpu.VMEM((1, _K), jnp.float32),
            pltpu.VMEM((1, _K), jnp.float32),
            pltpu.VMEM((8, _M), jnp.float32),
            pltpu.VMEM((8, _N), jnp.float32),
        ],
    )(idx2, backs_t, xg_r, yg_r, v_r, xg_c, yg_c, v_c, dots,
      row_sum, col_sum)


@jax.jit
def kernel(src_points, src_points_c, ref_points, ref_points_c, ref_feats_c,
           src_feats_c, gt_node_corr_indices, gt_node_corr_overlaps,
           src_back_indices, ref_back_indices):
    queries = jnp.stack([src_points_c, ref_points_c])
    pts = jnp.stack([src_points, ref_points])
    pts = jnp.pad(pts, ((0, 0), (0, _PPAD - _NPTS), (0, 0)),
                  constant_values=1e8)
    pts_t = jnp.transpose(pts, (0, 2, 1))
    idx2 = _nn_argmin(queries, pts_t)

    row_sum, col_sum = _dense_sums(ref_feats_c,
                                   jnp.transpose(src_feats_c, (1, 0)))

    backs = jnp.stack([src_back_indices, ref_back_indices])
    backs_t = jnp.pad(backs, ((0, 0), (0, _BPAD - backs.shape[1])),
                      constant_values=-1)[:, :, None]

    xg = gt_node_corr_indices[:, 0].astype(jnp.int32)
    yg = gt_node_corr_indices[:, 1].astype(jnp.int32)
    v = gt_node_corr_overlaps

    dots = _cand_dots_sc(ref_feats_c, src_feats_c, xg, yg)

    out = _combine(idx2, backs_t, xg[None, :], yg[None, :], v[None, :],
                   xg[:, None], yg[:, None], v[:, None],
                   dots[None, :], row_sum, col_sum)
    return (out[0, 0], out[0, 1], out[0, 2])


# speedup vs baseline: 1.6147x; 1.0211x over previous
import functools

import jax
import jax.numpy as jnp
from jax import lax
from jax.experimental import pallas as pl
from jax.experimental.pallas import tpu as pltpu
from jax.experimental.pallas import tpu_sc as plsc

_POS_MARGIN = 0.1
_NEG_MARGIN = 1.4
_LOG_SCALE = 24.0
_MAX_POINTS = 256
_M = 1024
_N = 1024
_K = 4096
_D = 512
_NPTS = 20000
_PPAD = 20480
_PCH = 2048
_NPCH = _PPAD // _PCH
_BPAD = 10240
_BCH = 2048
_CCH = 1024
_GCH = 512
_HI = jax.lax.Precision.HIGHEST


def _nn_body(q_ref, p_ref, idx_ref, run_v, run_i):
    ch = pl.program_id(1)
    q = q_ref[0]
    p = p_ref[0]
    a2 = jnp.sum(q * q, axis=1)[:, None]
    b2 = jnp.sum(p * p, axis=0)[None, :]
    s = a2 + b2 - 2.0 * jnp.dot(q, p, preferred_element_type=jnp.float32)
    gidx = ch * _PCH + jax.lax.broadcasted_iota(jnp.int32, (_M, _PCH), 1)

    @pl.when(ch == 0)
    def _():
        run_v[...] = s
        run_i[...] = gidx

    @pl.when(ch > 0)
    def _():
        rv = run_v[...]
        better = s < rv
        run_v[...] = jnp.where(better, s, rv)
        run_i[...] = jnp.where(better, gidx, run_i[...])

    @pl.when(ch == _NPCH - 1)
    def _():
        rv = jnp.sqrt(jnp.clip(run_v[...], 0.0, None))
        ri = run_i[...]
        m = jnp.min(rv, axis=1)
        mi = jnp.min(jnp.where(rv == m[:, None], ri, jnp.int32(2 ** 30)),
                     axis=1)
        idx_ref[0] = mi[None, :]


def _nn_argmin(queries, points_t):
    return pl.pallas_call(
        _nn_body,
        grid=(2, _NPCH),
        in_specs=[
            pl.BlockSpec((1, _M, 3), lambda g, c: (g, 0, 0)),
            pl.BlockSpec((1, 3, _PCH), lambda g, c: (g, 0, c)),
        ],
        out_specs=pl.BlockSpec((1, 1, _M), lambda g, c: (g, 0, 0)),
        out_shape=jax.ShapeDtypeStruct((2, 1, _M), jnp.int32),
        scratch_shapes=[
            pltpu.VMEM((_M, _PCH), jnp.float32),
            pltpu.VMEM((_M, _PCH), jnp.int32),
        ],
    )(queries, points_t)


def _dense_body(r_ref, st_ref, row_ref, col_ref):
    f = jnp.dot(r_ref[...], st_ref[...], preferred_element_type=jnp.float32)
    feat = jnp.sqrt(jnp.clip(2.0 - 2.0 * f, 0.0, None))
    e = jnp.exp(_LOG_SCALE * jnp.maximum(_NEG_MARGIN - feat, 0.0) ** 2)
    row_ref[...] = jnp.sum(e, axis=1)[None, :]
    col_ref[...] = jnp.sum(e, axis=0)[None, :]


def _dense_sums(ref_feats, src_feats_t):
    return pl.pallas_call(
        _dense_body,
        out_shape=[
            jax.ShapeDtypeStruct((1, _M), jnp.float32),
            jax.ShapeDtypeStruct((1, _N), jnp.float32),
        ],
    )(ref_feats, src_feats_t)


_NW = 32
_BPW = _K // _NW
_HB = 64


def _cand_dots_sc(rf, sf, xg, yg):
    mesh = plsc.VectorSubcoreMesh(core_axis_name="c", subcore_axis_name="s")

    @functools.partial(
        pl.kernel, mesh=mesh,
        out_type=jax.ShapeDtypeStruct((_K,), jnp.float32),
        scratch_types=[
            pltpu.VMEM((_BPW,), jnp.int32),
            pltpu.VMEM((_BPW,), jnp.int32),
            pltpu.VMEM((_HB, _D), jnp.float32),
            pltpu.VMEM((_HB, _D), jnp.float32),
            pltpu.VMEM((_BPW,), jnp.float32),
            pltpu.SMEM((_BPW,), jnp.float32),
            pltpu.SemaphoreType.DMA,
        ],
    )
    def k(rf_hbm, sf_hbm, xg_hbm, yg_hbm, out_hbm, xi, yi, rrows, srows,
          dots, dots_sm, sem):
        wid = lax.axis_index("s") * 2 + lax.axis_index("c")
        base = wid * _BPW
        lane = lax.iota(jnp.int32, 16)
        pltpu.sync_copy(xg_hbm.at[pl.ds(base, _BPW)], xi)
        pltpu.sync_copy(yg_hbm.at[pl.ds(base, _BPW)], yi)
        for h in range(_BPW // _HB):
            pltpu.async_copy(rf_hbm.at[xi.at[pl.ds(h * _HB, _HB)]], rrows,
                             sem).wait()
            pltpu.async_copy(sf_hbm.at[yi.at[pl.ds(h * _HB, _HB)]], srows,
                             sem).wait()

            def row(r, carry):
                acc = jnp.zeros((16,), jnp.float32)
                for kk in range(_D // 16):
                    acc = acc + (rrows[r, pl.ds(kk * 16, 16)] *
                                 srows[r, pl.ds(kk * 16, 16)])
                s = acc[0]
                for l in range(1, 16):
                    s = s + acc[l]
                dots_sm[h * _HB + r] = s
                return carry

            lax.fori_loop(0, _HB, row, 0)
        for g in range(_BPW // 16):
            vec = jnp.zeros((16,), jnp.float32)
            for l in range(16):
                vec = jnp.where(lane == l, dots_sm[g * 16 + l], vec)
            dots[g * 16:(g + 1) * 16] = vec
        pltpu.sync_copy(dots, out_hbm.at[pl.ds(base, _BPW)])

    return k(rf, sf, xg, yg)


def _combine_body(idx_ref_, back_ref, xgr, ygr, vr, xgc, ygc, vc,
                  dots_ref, row_ref, col_ref, out_ref,
                  act_s, sel_s, sp_s, sn_s, accr_s, accc_s):
    def mask_for(g):
        idx_row = idx_ref_[g, 0:1, :]

        def body(ch, acc):
            b = back_ref[g, pl.ds(ch * _BCH, _BCH), 0:1]
            hit = jnp.max(jnp.where(idx_row == b, 1.0, 0.0), axis=0,
                          keepdims=True)
            return jnp.maximum(acc, hit)

        return jax.lax.fori_loop(0, _BPAD // _BCH, body,
                                 jnp.zeros((1, _M), jnp.float32))

    mask_src_row = mask_for(0)
    mask_ref_row = mask_for(1)

    c_col = xgc[...] * _N + ygc[...]
    v_col = vc[...]
    ar_col = jax.lax.broadcasted_iota(jnp.int32, (_K, 1), 0)
    riota_col = jax.lax.broadcasted_iota(jnp.int32, (_M, 1), 0)

    def l1(i, carry):
        sl = pl.ds(i * _CCH, _CCH)
        xc_row = xgr[0:1, sl]
        yc_row = ygr[0:1, sl]
        cc_row = xc_row * _N + yc_row
        ii_row = i * _CCH + jax.lax.broadcasted_iota(jnp.int32, (1, _CCH), 1)
        dup = jnp.max(jnp.where((c_col == cc_row) & (ar_col > ii_row),
                                1.0, 0.0), axis=0, keepdims=True)
        ohx_t = jnp.where(riota_col == xc_row, 1.0, 0.0)
        ohy_t = jnp.where(riota_col == yc_row, 1.0, 0.0)
        mr = jnp.dot(mask_ref_row, ohx_t, preferred_element_type=jnp.float32)
        ms = jnp.dot(mask_src_row, ohy_t, preferred_element_type=jnp.float32)
        act_s[0:1, sl] = jnp.where((mr > 0.5) & (ms > 0.5) & (dup < 0.5),
                                   1.0, 0.0)
        return carry

    jax.lax.fori_loop(0, _K // _CCH, l1, 0)
    act_row = act_s[...]
    count = jnp.sum(act_row)

    def l2(i, carry):
        sl = pl.ds(i * _CCH, _CCH)
        vc_row = vr[0:1, sl]
        cc_row = xgr[0:1, sl] * _N + ygr[0:1, sl]
        beats = jnp.where((v_col > vc_row) |
                          ((v_col == vc_row) & (c_col < cc_row)), 1.0, 0.0)
        rank = jnp.dot(act_row, beats, preferred_element_type=jnp.float32)
        actc = act_s[0:1, sl]
        sel_s[0:1, sl] = jnp.where(
            (actc > 0.5) & ((count <= float(_MAX_POINTS)) |
                            (rank < float(_MAX_POINTS))), 1.0, 0.0)
        return carry

    jax.lax.fori_loop(0, _K // _CCH, l2, 0)

    riota_row = jax.lax.broadcasted_iota(jnp.int32, (1, _M), 1)
    fc = jnp.sqrt(jnp.clip(2.0 - 2.0 * dots_ref[...], 0.0, None))
    sp_s[...] = _LOG_SCALE * jnp.maximum(fc - _POS_MARGIN, 0.0) ** 2
    sn_s[...] = _LOG_SCALE * jnp.maximum(_NEG_MARGIN - fc, 0.0) ** 2

    sel_row = sel_s[...]
    sp_row = sp_s[...]
    g_shift = jnp.max(jnp.where(sel_row > 0.5, sp_row, 0.0))

    accr_s[...] = jnp.zeros((8, _M), jnp.float32)
    accc_s[...] = jnp.zeros((8, _N), jnp.float32)

    def l4(i, carry):
        sl = pl.ds(i * _GCH, _GCH)
        selc = sel_s[0:1, sl]
        spc = sp_s[0:1, sl]
        snc = sn_s[0:1, sl]
        wp = jnp.exp(jnp.where(selc > 0.5, spc - g_shift, -1e30))
        wn = selc * jnp.exp(snc)
        w = jnp.concatenate(
            [selc, wp, wn, jnp.zeros((5, _GCH), jnp.float32)], axis=0)
        xc_col = xgc[sl, 0:1]
        yc_col = ygc[sl, 0:1]
        ohx = jnp.where(xc_col == riota_row, 1.0, 0.0)
        ohy = jnp.where(yc_col == riota_row, 1.0, 0.0)
        accr_s[...] = accr_s[...] + jnp.dot(
            w, ohx, preferred_element_type=jnp.float32, precision=_HI)
        accc_s[...] = accc_s[...] + jnp.dot(
            w, ohy, preferred_element_type=jnp.float32, precision=_HI)
        return carry

    jax.lax.fori_loop(0, _K // _GCH, l4, 0)

    npos_r, sp_r, sn_r = accr_s[0:1, :], accr_s[1:2, :], accr_s[2:3, :]
    npos_c, sp_c, sn_c = accc_s[0:1, :], accc_s[1:2, :], accc_s[2:3, :]

    row_sum = row_ref[...]
    col_sum = col_ref[...]
    eg = jnp.exp(-g_shift)
    lse_neg_r = jnp.log(row_sum - sn_r + npos_r)
    lse_pos_r = g_shift + jnp.log((float(_N) - npos_r) * eg + sp_r)
    lse_neg_c = jnp.log(col_sum - sn_c + npos_c)
    lse_pos_c = g_shift + jnp.log((float(_M) - npos_c) * eg + sp_c)

    def softplus(x):
        return jnp.maximum(x, 0.0) + jnp.log(1.0 + jnp.exp(-jnp.abs(x)))

    loss_row = softplus(lse_pos_r + lse_neg_r) / _LOG_SCALE
    loss_col = softplus(lse_pos_c + lse_neg_c) / _LOG_SCALE
    loss1 = (jnp.sum(loss_row) / float(_M) + jnp.sum(loss_col) / float(_N)) / 2.0
    loss2 = (jnp.sum(mask_ref_row) + jnp.sum(mask_src_row)) / float(_M + _N)
    loss = loss1 + loss2

    oi = jax.lax.broadcasted_iota(jnp.int32, (1, 128), 1)
    out_ref[...] = jnp.where(oi == 0, loss,
                             jnp.where(oi == 1, loss1,
                                       jnp.where(oi == 2, loss2, 0.0)))


def _combine(idx2, backs_t, xg_r, yg_r, v_r, xg_c, yg_c, v_c, dots,
             row_sum, col_sum):
    return pl.pallas_call(
        _combine_body,
        out_shape=jax.ShapeDtypeStruct((1, 128), jnp.float32),
        scratch_shapes=[
            pltpu.VMEM((1, _K), jnp.float32),
            pltpu.VMEM((1, _K), jnp.float32),
            pltpu.VMEM((1, _K), jnp.float32),
            pltpu.VMEM((1, _K), jnp.float32),
            pltpu.VMEM((8, _M), jnp.float32),
            pltpu.VMEM((8, _N), jnp.float32),
        ],
    )(idx2, backs_t, xg_r, yg_r, v_r, xg_c, yg_c, v_c, dots,
      row_sum, col_sum)


@jax.jit
def kernel(src_points, src_points_c, ref_points, ref_points_c, ref_feats_c,
           src_feats_c, gt_node_corr_indices, gt_node_corr_overlaps,
           src_back_indices, ref_back_indices):
    queries = jnp.stack([src_points_c, ref_points_c])
    pts = jnp.stack([src_points, ref_points])
    pts = jnp.pad(pts, ((0, 0), (0, _PPAD - _NPTS), (0, 0)),
                  constant_values=1e8)
    pts_t = jnp.transpose(pts, (0, 2, 1))
    idx2 = _nn_argmin(queries, pts_t)

    row_sum, col_sum = _dense_sums(ref_feats_c,
                                   jnp.transpose(src_feats_c, (1, 0)))

    backs = jnp.stack([src_back_indices, ref_back_indices])
    backs_t = jnp.pad(backs, ((0, 0), (0, _BPAD - backs.shape[1])),
                      constant_values=-1)[:, :, None]

    xg = gt_node_corr_indices[:, 0].astype(jnp.int32)
    yg = gt_node_corr_indices[:, 1].astype(jnp.int32)
    v = gt_node_corr_overlaps

    dots = _cand_dots_sc(ref_feats_c, src_feats_c, xg, yg)

    out = _combine(idx2, backs_t, xg[None, :], yg[None, :], v[None, :],
                   xg[:, None], yg[:, None], v[:, None],
                   dots[None, :], row_sum, col_sum)
    return (out[0, 0], out[0, 1], out[0, 2])
